# baseline (speedup 1.0000x reference)
import functools

import jax
import jax.numpy as jnp
from jax import lax
from jax.experimental import pallas as pl
from jax.experimental.pallas import tpu as pltpu
from jax.experimental.pallas import tpu_sc as plsc

NC = 2
NS = 16
CHUNK = 32


def _matmul(X, W):
    n, d_in = X.shape
    d_out = W.shape[1]
    bm = 2000 if n % 2000 == 0 else n
    grid = (n // bm,)

    def mm_body(x_ref, w_ref, o_ref):
        o_ref[...] = jnp.dot(x_ref[...], w_ref[...],
                             preferred_element_type=jnp.float32)

    return pl.pallas_call(
        mm_body,
        grid=grid,
        in_specs=[
            pl.BlockSpec((bm, d_in), lambda i: (i, 0)),
            pl.BlockSpec((d_in, d_out), lambda i: (0, 0)),
        ],
        out_specs=pl.BlockSpec((bm, d_out), lambda i: (i, 0)),
        out_shape=jax.ShapeDtypeStruct((n, d_out), jnp.float32),
    )(X, W)


def _add(p, n_out):
    d = p.shape[2]
    bm = 2000 if n_out % 2000 == 0 else n_out
    grid = (n_out // bm,)

    def add_body(a_ref, b_ref, o_ref):
        o_ref[...] = a_ref[0] + b_ref[0]

    return pl.pallas_call(
        add_body,
        grid=grid,
        in_specs=[
            pl.BlockSpec((1, bm, d), lambda i: (0, i, 0)),
            pl.BlockSpec((1, bm, d), lambda i: (1, i, 0)),
        ],
        out_specs=pl.BlockSpec((bm, d), lambda i: (i, 0)),
        out_shape=jax.ShapeDtypeStruct((n_out, d), jnp.float32),
    )(p, p)


NBUF = 8
GC = 16


def _make_sc_scatter(n_acc, d, cpw):
    mesh = plsc.VectorSubcoreMesh(core_axis_name="c", subcore_axis_name="s",
                                  num_cores=NC, num_subcores=NS)
    zpt = n_acc // (NS * CHUNK)
    groups = cpw // GC

    nch = NC * NS * cpw

    @functools.partial(
        pl.kernel,
        out_type=jax.ShapeDtypeStruct((NC * n_acc, d), jnp.float32),
        mesh=mesh,
        scratch_types=[
            [pltpu.VMEM((GC, CHUNK), jnp.int32) for _ in range(2)],
            [pltpu.VMEM((GC, CHUNK), jnp.int32) for _ in range(2)],
            [pltpu.VMEM((CHUNK, d), jnp.float32) for _ in range(NBUF)],
            pltpu.VMEM_SHARED((n_acc, d), jnp.float32),
            [pltpu.SemaphoreType.DMA for _ in range(NBUF)],
            [pltpu.SemaphoreType.DMA for _ in range(NBUF)],
            [pltpu.SemaphoreType.DMA for _ in range(2)],
        ],
    )
    def sc_kernel(comb_hbm, ei_hbm, out_hbm,
                  icol, irow, bufs, acc_sh, gsems, ssems, isems):
        c = lax.axis_index("c")
        s = lax.axis_index("s")
        wid = c * NS + s

        sc_zero = jax.named_scope("sc_zero")
        sc_zero.__enter__()

        def zrow(i, carry):
            for j in range(d // 16):
                bufs[0][i, pl.ds(j * 16, 16)] = jnp.zeros((16,), jnp.float32)
            return carry
        lax.fori_loop(0, CHUNK, zrow, 0)

        pltpu.async_copy(ei_hbm.at[pl.ds(nch + wid * cpw, GC)], icol[0],
                         isems[0])
        pltpu.async_copy(ei_hbm.at[pl.ds(wid * cpw, GC)], irow[0], isems[0])

        def zcopy(j, carry):
            base = (s * zpt + j) * CHUNK
            pltpu.sync_copy(bufs[0], acc_sh.at[pl.ds(base, CHUNK)])
            return carry
        lax.fori_loop(0, zpt, zcopy, 0)
        plsc.subcore_barrier()
        sc_zero.__exit__(None, None, None)

        sc_edges = jax.named_scope("sc_edges")
        sc_edges.__enter__()
        for g in range(groups):
            ib = g % 2
            ic, ir = icol[ib], irow[ib]
            pltpu.make_async_copy(ei_hbm.at[pl.ds(0, GC)], ic,
                                  isems[ib]).wait()
            pltpu.make_async_copy(ei_hbm.at[pl.ds(0, GC)], ir,
                                  isems[ib]).wait()
            if g + 1 < groups:
                nb = (g + 1) % 2
                base = wid * cpw + (g + 1) * GC
                pltpu.async_copy(ei_hbm.at[pl.ds(nch + base, GC)], icol[nb],
                                 isems[nb])
                pltpu.async_copy(ei_hbm.at[pl.ds(base, GC)], irow[nb],
                                 isems[nb])

            for b in range(NBUF):
                pltpu.async_copy(comb_hbm.at[ic.at[b]], bufs[b], gsems[b])

            def ring(r, carry):
                k = r * NBUF
                for b in range(NBUF):
                    pltpu.make_async_copy(comb_hbm.at[ic.at[k + b]],
                                          bufs[b], gsems[b]).wait()
                    pltpu.async_copy(bufs[b], acc_sh.at[ir.at[k + b]],
                                     ssems[b], add=True)
                for b in range(NBUF):
                    pltpu.make_async_copy(bufs[b], acc_sh.at[ir.at[k + b]],
                                          ssems[b]).wait()
                    pltpu.async_copy(comb_hbm.at[ic.at[k + NBUF + b]],
                                     bufs[b], gsems[b])
                return carry
            lax.fori_loop(0, GC // NBUF - 1, ring, 0)
            kl = GC - NBUF
            for b in range(NBUF):
                pltpu.make_async_copy(comb_hbm.at[ic.at[kl + b]],
                                      bufs[b], gsems[b]).wait()
                pltpu.async_copy(bufs[b], acc_sh.at[ir.at[kl + b]],
                                 ssems[b], add=True)
            for b in range(NBUF):
                pltpu.make_async_copy(bufs[b], acc_sh.at[ir.at[kl + b]],
                                      ssems[b]).wait()
        plsc.subcore_barrier()
        sc_edges.__exit__(None, None, None)

        with jax.named_scope("sc_wout"):
            def wout(j, carry):
                base = (s * zpt + j) * CHUNK
                pltpu.sync_copy(acc_sh.at[pl.ds(base, CHUNK)],
                                out_hbm.at[pl.ds(c * n_acc + base, CHUNK)])
                return carry
            lax.fori_loop(0, zpt, wout, 0)

    return sc_kernel


@jax.jit
def kernel(X, edge_index, W):
    n, _ = X.shape
    d = W.shape[1]
    e = edge_index.shape[1]

    nw = NC * NS
    cpw = -(-e // (CHUNK * nw))
    cpw = -(-cpw // GC) * GC
    e_pad = cpw * nw * CHUNK
    n_acc = -(-(n + 1) // (NS * CHUNK)) * (NS * CHUNK)

    pad = e_pad - e
    ei = edge_index
    if pad:
        spread = jnp.arange(pad, dtype=jnp.int32)
        ei = jnp.concatenate(
            [ei, jnp.stack([n + spread % (n_acc - n), spread % n])], axis=1)
    ei = ei.reshape(2 * nw * cpw, CHUNK)

    comb = _matmul(X, W)
    partials = _make_sc_scatter(n_acc, d, cpw)(comb, ei)
    return _add(partials.reshape(NC, n_acc, d), n)


# speedup vs baseline: 1.1593x; 1.1593x over previous
import functools

import jax
import jax.numpy as jnp
from jax import lax
from jax.experimental import pallas as pl
from jax.experimental.pallas import tpu as pltpu
from jax.experimental.pallas import tpu_sc as plsc

NC = 2
NS = 16
CHUNK = 64


def _matmul(X, W):
    n, d_in = X.shape
    d_out = W.shape[1]
    bm = 2000 if n % 2000 == 0 else n
    grid = (n // bm,)

    def mm_body(x_ref, w_ref, o_ref):
        o_ref[...] = jnp.dot(x_ref[...], w_ref[...],
                             preferred_element_type=jnp.float32)

    return pl.pallas_call(
        mm_body,
        grid=grid,
        in_specs=[
            pl.BlockSpec((bm, d_in), lambda i: (i, 0)),
            pl.BlockSpec((d_in, d_out), lambda i: (0, 0)),
        ],
        out_specs=pl.BlockSpec((bm, d_out), lambda i: (i, 0)),
        out_shape=jax.ShapeDtypeStruct((n, d_out), jnp.float32),
    )(X, W)


def _add(p, n_out):
    d = p.shape[2]
    bm = 2000 if n_out % 2000 == 0 else n_out
    grid = (n_out // bm,)

    def add_body(a_ref, b_ref, o_ref):
        o_ref[...] = a_ref[0] + b_ref[0]

    return pl.pallas_call(
        add_body,
        grid=grid,
        in_specs=[
            pl.BlockSpec((1, bm, d), lambda i: (0, i, 0)),
            pl.BlockSpec((1, bm, d), lambda i: (1, i, 0)),
        ],
        out_specs=pl.BlockSpec((bm, d), lambda i: (i, 0)),
        out_shape=jax.ShapeDtypeStruct((n_out, d), jnp.float32),
    )(p, p)


NBUF = 4
GC = 32


def _make_sc_scatter(n_acc, d, cpw):
    mesh = plsc.VectorSubcoreMesh(core_axis_name="c", subcore_axis_name="s",
                                  num_cores=NC, num_subcores=NS)
    zpt = n_acc // (NS * CHUNK)
    groups = cpw // GC

    nch = NC * NS * cpw

    @functools.partial(
        pl.kernel,
        out_type=jax.ShapeDtypeStruct((NC * n_acc, d), jnp.float32),
        mesh=mesh,
        scratch_types=[
            [pltpu.VMEM((GC, CHUNK), jnp.int32) for _ in range(2)],
            [pltpu.VMEM((GC, CHUNK), jnp.int32) for _ in range(2)],
            [pltpu.VMEM((CHUNK, d), jnp.float32) for _ in range(NBUF)],
            pltpu.VMEM_SHARED((n_acc, d), jnp.float32),
            [pltpu.SemaphoreType.DMA for _ in range(NBUF)],
            [pltpu.SemaphoreType.DMA for _ in range(NBUF)],
            [pltpu.SemaphoreType.DMA for _ in range(2)],
        ],
    )
    def sc_kernel(comb_hbm, ei_hbm, out_hbm,
                  icol, irow, bufs, acc_sh, gsems, ssems, isems):
        c = lax.axis_index("c")
        s = lax.axis_index("s")
        wid = c * NS + s

        sc_zero = jax.named_scope("sc_zero")
        sc_zero.__enter__()

        def zrow(i, carry):
            for j in range(d // 16):
                bufs[0][i, pl.ds(j * 16, 16)] = jnp.zeros((16,), jnp.float32)
            return carry
        lax.fori_loop(0, CHUNK, zrow, 0)

        pltpu.async_copy(ei_hbm.at[pl.ds(nch + wid * cpw, GC)], icol[0],
                         isems[0])
        pltpu.async_copy(ei_hbm.at[pl.ds(wid * cpw, GC)], irow[0], isems[0])

        def zcopy(j, carry):
            base = (s * zpt + j) * CHUNK
            pltpu.sync_copy(bufs[0], acc_sh.at[pl.ds(base, CHUNK)])
            return carry
        lax.fori_loop(0, zpt, zcopy, 0)
        plsc.subcore_barrier()
        sc_zero.__exit__(None, None, None)

        sc_edges = jax.named_scope("sc_edges")
        sc_edges.__enter__()
        for g in range(groups):
            ib = g % 2
            ic, ir = icol[ib], irow[ib]
            pltpu.make_async_copy(ei_hbm.at[pl.ds(0, GC)], ic,
                                  isems[ib]).wait()
            pltpu.make_async_copy(ei_hbm.at[pl.ds(0, GC)], ir,
                                  isems[ib]).wait()
            if g + 1 < groups:
                nb = (g + 1) % 2
                base = wid * cpw + (g + 1) * GC
                pltpu.async_copy(ei_hbm.at[pl.ds(nch + base, GC)], icol[nb],
                                 isems[nb])
                pltpu.async_copy(ei_hbm.at[pl.ds(base, GC)], irow[nb],
                                 isems[nb])

            for b in range(NBUF):
                pltpu.async_copy(comb_hbm.at[ic.at[b]], bufs[b], gsems[b])

            def ring(r, carry):
                k = r * NBUF
                for b in range(NBUF):
                    pltpu.make_async_copy(comb_hbm.at[ic.at[k + b]],
                                          bufs[b], gsems[b]).wait()
                    pltpu.async_copy(bufs[b], acc_sh.at[ir.at[k + b]],
                                     ssems[b], add=True)
                for b in range(NBUF):
                    pltpu.make_async_copy(bufs[b], acc_sh.at[ir.at[k + b]],
                                          ssems[b]).wait()
                    pltpu.async_copy(comb_hbm.at[ic.at[k + NBUF + b]],
                                     bufs[b], gsems[b])
                return carry
            lax.fori_loop(0, GC // NBUF - 1, ring, 0)
            kl = GC - NBUF
            for b in range(NBUF):
                pltpu.make_async_copy(comb_hbm.at[ic.at[kl + b]],
                                      bufs[b], gsems[b]).wait()
                pltpu.async_copy(bufs[b], acc_sh.at[ir.at[kl + b]],
                                 ssems[b], add=True)
            for b in range(NBUF):
                pltpu.make_async_copy(bufs[b], acc_sh.at[ir.at[kl + b]],
                                      ssems[b]).wait()
        plsc.subcore_barrier()
        sc_edges.__exit__(None, None, None)

        with jax.named_scope("sc_wout"):
            def wout(j, carry):
                base = (s * zpt + j) * CHUNK
                pltpu.sync_copy(acc_sh.at[pl.ds(base, CHUNK)],
                                out_hbm.at[pl.ds(c * n_acc + base, CHUNK)])
                return carry
            lax.fori_loop(0, zpt, wout, 0)

    return sc_kernel


@jax.jit
def kernel(X, edge_index, W):
    n, _ = X.shape
    d = W.shape[1]
    e = edge_index.shape[1]

    nw = NC * NS
    cpw = -(-e // (CHUNK * nw))
    cpw = -(-cpw // GC) * GC
    e_pad = cpw * nw * CHUNK
    n_acc = -(-(n + 1) // (NS * CHUNK)) * (NS * CHUNK)

    pad = e_pad - e
    ei = edge_index
    if pad:
        spread = jnp.arange(pad, dtype=jnp.int32)
        ei = jnp.concatenate(
            [ei, jnp.stack([n + spread % (n_acc - n), spread % n])], axis=1)
    ei = ei.reshape(2 * nw * cpw, CHUNK)

    comb = _matmul(X, W)
    partials = _make_sc_scatter(n_acc, d, cpw)(comb, ei)
    return _add(partials.reshape(NC, n_acc, d), n)
